# baseline (speedup 1.0000x reference)
import functools
import math

import jax
import jax.numpy as jnp
from jax import lax
from jax.experimental import pallas as pl
from jax.experimental.pallas import tpu as pltpu
from jax.experimental.pallas import tpu_sc as plsc

B = 4
S = 4096
D = 1024
SCALE = math.sqrt(D)

NC = 2
NS = 16
NW = NC * NS
ROWS_PER_W = (B * S) // NW
W_PER_B = NW // B
CH = 16
NCHUNK = ROWS_PER_W // CH
G = D // 16

_mesh = plsc.VectorSubcoreMesh(
    core_axis_name="c", subcore_axis_name="s", num_cores=NC, num_subcores=NS
)


@functools.partial(
    pl.kernel,
    out_type=(
        jax.ShapeDtypeStruct((B, S, D), jnp.float32),
        jax.ShapeDtypeStruct((B, W_PER_B, D), jnp.float32),
    ),
    mesh=_mesh,
    scratch_types=[
        pltpu.VMEM((ROWS_PER_W,), jnp.int32),
        pltpu.VMEM((CH, D), jnp.float32),
        pltpu.VMEM((CH, D), jnp.float32),
        pltpu.VMEM((D,), jnp.float32),
        pltpu.VMEM((D,), jnp.float32),
        pltpu.SemaphoreType.DMA,
        pltpu.SemaphoreType.DMA,
    ],
)
def _sc_embed(ids_hbm, tok_hbm, pos_hbm, mod_hbm,
              out_hbm, part_hbm,
              idx_v, tok_v, pos_v, mod_v, acc_v, sem1, sem2):
    wid = lax.axis_index("s") * NC + lax.axis_index("c")
    b = wid // W_PER_B
    sub = wid % W_PER_B
    seq0 = sub * ROWS_PER_W

    pltpu.sync_copy(ids_hbm.at[b, pl.ds(seq0, ROWS_PER_W)], idx_v)
    pltpu.sync_copy(mod_hbm, mod_v)

    def zero_body(g, _):
        acc_v[pl.ds(g * 16, 16)] = jnp.zeros((16,), jnp.float32)
        return 0
    lax.fori_loop(0, G, zero_body, 0)

    def chunk_body(c, _):
        s0 = c * CH
        cp1 = pltpu.async_copy(tok_hbm.at[idx_v.at[pl.ds(s0, CH)]], tok_v, sem1)
        cp2 = pltpu.async_copy(pos_hbm.at[pl.ds(seq0 + s0, CH)], pos_v, sem2)
        cp1.wait()
        cp2.wait()

        def row_body(r, _):
            def g_body(g, _):
                o = g * 16
                t = tok_v[r, pl.ds(o, 16)]
                p = pos_v[r, pl.ds(o, 16)]
                m = mod_v[pl.ds(o, 16)]
                e = t * SCALE + p + m
                tok_v[r, pl.ds(o, 16)] = e
                plsc.addupdate(acc_v.at[pl.ds(o, 16)], e)
                return 0
            lax.fori_loop(0, G, g_body, 0)
            return 0
        lax.fori_loop(0, CH, row_body, 0)

        pltpu.sync_copy(tok_v, out_hbm.at[b, pl.ds(seq0 + s0, CH)])
        return 0
    lax.fori_loop(0, NCHUNK, chunk_body, 0)

    pltpu.sync_copy(acc_v, part_hbm.at[b, sub])


def _ctx_body(part_ref, w_ref, b_ref, out_ref):
    mean = jnp.sum(part_ref[...], axis=1) * (1.0 / S)
    out_ref[...] = (
        jnp.dot(mean, w_ref[...], preferred_element_type=jnp.float32)
        + b_ref[...]
    )


_ctx_proj = pl.pallas_call(
    _ctx_body,
    out_shape=jax.ShapeDtypeStruct((B, D), jnp.float32),
)


def kernel(input_ids, modality, token_embed, pos_embed, modality_embed, W_ctx, b_ctx):
    ids = input_ids.astype(jnp.int32)
    mod_row = lax.dynamic_index_in_dim(
        modality_embed, modality, axis=0, keepdims=False
    )
    emb, part = _sc_embed(ids, token_embed, pos_embed, mod_row)
    context = _ctx_proj(part, W_ctx, b_ctx.reshape(1, D))
    return emb, context


# speedup vs baseline: 2.8426x; 2.8426x over previous
import functools
import math

import jax
import jax.numpy as jnp
from jax import lax
from jax.experimental import pallas as pl
from jax.experimental.pallas import tpu as pltpu
from jax.experimental.pallas import tpu_sc as plsc

B = 4
S = 4096
D = 1024
SCALE = math.sqrt(D)

NC = 2
NS = 16
NW = NC * NS
ROWS_PER_W = (B * S) // NW
W_PER_B = NW // B
CH = 16
NCHUNK = ROWS_PER_W // CH
NPAIR = NCHUNK // 2
G = D // 16

_mesh = plsc.VectorSubcoreMesh(
    core_axis_name="c", subcore_axis_name="s", num_cores=NC, num_subcores=NS
)


@functools.partial(
    pl.kernel,
    out_type=(
        jax.ShapeDtypeStruct((B, S, D), jnp.float32),
        jax.ShapeDtypeStruct((B, W_PER_B, D), jnp.float32),
    ),
    mesh=_mesh,
    scratch_types=[
        pltpu.VMEM((ROWS_PER_W,), jnp.int32),
        pltpu.VMEM((CH, D), jnp.float32),
        pltpu.VMEM((CH, D), jnp.float32),
        pltpu.VMEM((CH, D), jnp.float32),
        pltpu.VMEM((CH, D), jnp.float32),
        pltpu.VMEM((CH, D), jnp.float32),
        pltpu.VMEM((CH, D), jnp.float32),
        pltpu.VMEM((D,), jnp.float32),
        pltpu.VMEM((D,), jnp.float32),
        pltpu.SemaphoreType.DMA,
        pltpu.SemaphoreType.DMA,
        pltpu.SemaphoreType.DMA,
        pltpu.SemaphoreType.DMA,
        pltpu.SemaphoreType.DMA,
        pltpu.SemaphoreType.DMA,
    ],
)
def _sc_embed(ids_hbm, tok_hbm, pos_hbm, mod_hbm,
              out_hbm, part_hbm,
              idx_v, tok0, tok1, pos0, pos1, o0, o1, mod_v, acc_v,
              gt0, gt1, gp0, gp1, go0, go1):
    wid = lax.axis_index("s") * NC + lax.axis_index("c")
    b = wid // W_PER_B
    sub = wid % W_PER_B
    seq0 = sub * ROWS_PER_W

    pltpu.sync_copy(ids_hbm.at[b, pl.ds(seq0, ROWS_PER_W)], idx_v)
    pltpu.sync_copy(mod_hbm, mod_v)

    def zero_body(g, _):
        acc_v[pl.ds(g * 16, 16)] = jnp.zeros((16,), jnp.float32)
        return 0
    lax.fori_loop(0, G, zero_body, 0)

    def tok_cp(c, buf, sem):
        return pltpu.make_async_copy(
            tok_hbm.at[idx_v.at[pl.ds(c * CH, CH)]], buf, sem)

    def pos_cp(c, buf, sem):
        return pltpu.make_async_copy(
            pos_hbm.at[pl.ds(seq0 + c * CH, CH)], buf, sem)

    def out_cp(c, buf, sem):
        return pltpu.make_async_copy(
            buf, out_hbm.at[b, pl.ds(seq0 + c * CH, CH)], sem)

    def compute_chunk(tok_v, pos_v, out_v):
        def g_body(g, _):
            o = g * 16
            m = mod_v[pl.ds(o, 16)]
            parts = [None, None, None, None]
            for r in range(CH):
                t = tok_v[r, pl.ds(o, 16)]
                p = pos_v[r, pl.ds(o, 16)]
                e = t * SCALE + p + m
                out_v[r, pl.ds(o, 16)] = e
                k = r % 4
                parts[k] = e if parts[k] is None else parts[k] + e
            acc_v[pl.ds(o, 16)] = (
                acc_v[pl.ds(o, 16)]
                + ((parts[0] + parts[1]) + (parts[2] + parts[3]))
            )
            return 0
        lax.fori_loop(0, G, g_body, 0)

    tok_cp(0, tok0, gt0).start()
    pos_cp(0, pos0, gp0).start()

    def pair_body(p, _):
        c0 = 2 * p
        tok_cp(c0 + 1, tok1, gt1).start()
        pos_cp(c0 + 1, pos1, gp1).start()
        tok_cp(c0, tok0, gt0).wait()
        pos_cp(c0, pos0, gp0).wait()
        @pl.when(p > 0)
        def _():
            out_cp(c0 - 2, o0, go0).wait()
        compute_chunk(tok0, pos0, o0)
        out_cp(c0, o0, go0).start()
        @pl.when(p < NPAIR - 1)
        def _():
            tok_cp(c0 + 2, tok0, gt0).start()
            pos_cp(c0 + 2, pos0, gp0).start()
        tok_cp(c0 + 1, tok1, gt1).wait()
        pos_cp(c0 + 1, pos1, gp1).wait()
        @pl.when(p > 0)
        def _():
            out_cp(c0 - 1, o1, go1).wait()
        compute_chunk(tok1, pos1, o1)
        out_cp(c0 + 1, o1, go1).start()
        return 0
    lax.fori_loop(0, NPAIR, pair_body, 0)

    out_cp(NCHUNK - 2, o0, go0).wait()
    out_cp(NCHUNK - 1, o1, go1).wait()

    pltpu.sync_copy(acc_v, part_hbm.at[b, sub])


def _ctx_body(part_ref, w_ref, b_ref, out_ref):
    mean = jnp.sum(part_ref[...], axis=1) * (1.0 / S)
    out_ref[...] = (
        jnp.dot(mean, w_ref[...], preferred_element_type=jnp.float32)
        + b_ref[...]
    )


_ctx_proj = pl.pallas_call(
    _ctx_body,
    out_shape=jax.ShapeDtypeStruct((B, D), jnp.float32),
)


def kernel(input_ids, modality, token_embed, pos_embed, modality_embed, W_ctx, b_ctx):
    ids = input_ids.astype(jnp.int32)
    mod_row = lax.dynamic_index_in_dim(
        modality_embed, modality, axis=0, keepdims=False
    )
    emb, part = _sc_embed(ids, token_embed, pos_embed, mod_row)
    context = _ctx_proj(part, W_ctx, b_ctx.reshape(1, D))
    return emb, context


# speedup vs baseline: 3.9022x; 1.3727x over previous
import functools
import math

import jax
import jax.numpy as jnp
from jax import lax
from jax.experimental import pallas as pl
from jax.experimental.pallas import tpu as pltpu
from jax.experimental.pallas import tpu_sc as plsc

B = 4
S = 4096
D = 1024
SCALE = math.sqrt(D)

NC = 2
NS = 16
NW = NC * NS
POS_PER_W = S // NW
CH = 8
NCHUNK = POS_PER_W // CH
G = D // 16

_mesh = plsc.VectorSubcoreMesh(
    core_axis_name="c", subcore_axis_name="s", num_cores=NC, num_subcores=NS
)


@functools.partial(
    pl.kernel,
    out_type=(
        jax.ShapeDtypeStruct((B, S, D), jnp.float32),
        jax.ShapeDtypeStruct((B, NW, D), jnp.float32),
    ),
    mesh=_mesh,
    scratch_types=[
        pltpu.VMEM((B, POS_PER_W), jnp.int32),
        pltpu.VMEM((B, CH, D), jnp.float32),
        pltpu.VMEM((B, CH, D), jnp.float32),
        pltpu.VMEM((B, CH, D), jnp.float32),
        pltpu.VMEM((CH, D), jnp.float32),
        pltpu.VMEM((CH, D), jnp.float32),
        pltpu.VMEM((D,), jnp.float32),
        pltpu.VMEM((B, D), jnp.float32),
        pltpu.SemaphoreType.DMA,
        pltpu.SemaphoreType.DMA,
        pltpu.SemaphoreType.DMA,
        pltpu.SemaphoreType.DMA,
        pltpu.SemaphoreType.DMA,
        pltpu.SemaphoreType.DMA,
        pltpu.SemaphoreType.DMA,
        pltpu.SemaphoreType.DMA,
    ],
)
def _sc_embed(ids_hbm, tok_hbm, pos_hbm, mod_hbm,
              out_hbm, part_hbm,
              idx_v, tokA, tokB, tokC, pos0, pos1, mod_v, acc_v,
              gtA, gtB, gtC, gp0, gp1, goA, goB, goC):
    wid = lax.axis_index("s") * NC + lax.axis_index("c")
    seq0 = wid * POS_PER_W

    toks = (tokA, tokB, tokC)
    gts = (gtA, gtB, gtC)
    gos = (goA, goB, goC)
    poss = (pos0, pos1)
    gps = (gp0, gp1)

    for bi in range(B):
        pltpu.sync_copy(ids_hbm.at[bi, pl.ds(seq0, POS_PER_W)], idx_v.at[bi])
    pltpu.sync_copy(mod_hbm, mod_v)

    def zero_body(j, _):
        acc_v[j // G, pl.ds((j % G) * 16, 16)] = jnp.zeros((16,), jnp.float32)
        return 0
    lax.fori_loop(0, B * G, zero_body, 0)

    def gather_cps(c, k):
        buf, sem = toks[k], gts[k]
        return [
            pltpu.make_async_copy(
                tok_hbm.at[idx_v.at[bi, pl.ds(c * CH, CH)]], buf.at[bi], sem)
            for bi in range(B)
        ]

    def pos_cp(c, j):
        return pltpu.make_async_copy(
            pos_hbm.at[pl.ds(seq0 + c * CH, CH)], poss[j], gps[j])

    def out_cp(c, k):
        return pltpu.make_async_copy(
            toks[k], out_hbm.at[:, pl.ds(seq0 + c * CH, CH)], gos[k])

    def compute_chunk(tok_v, pos_v):
        def g_body(g, _):
            o = g * 16
            m = mod_v[pl.ds(o, 16)]
            parts = [[None, None] for _ in range(B)]
            for r in range(CH):
                p = pos_v[r, pl.ds(o, 16)]
                pm = p + m
                for bi in range(B):
                    t = tok_v[bi, r, pl.ds(o, 16)]
                    e = t * SCALE + pm
                    tok_v[bi, r, pl.ds(o, 16)] = e
                    k = r % 2
                    pb = parts[bi]
                    pb[k] = e if pb[k] is None else pb[k] + e
            for bi in range(B):
                acc_v[bi, pl.ds(o, 16)] = (
                    acc_v[bi, pl.ds(o, 16)] + (parts[bi][0] + parts[bi][1])
                )
            return 0
        lax.fori_loop(0, G, g_body, 0)

    for cp in gather_cps(0, 0):
        cp.start()
    pos_cp(0, 0).start()
    for cp in gather_cps(1, 1):
        cp.start()
    pos_cp(1, 1).start()

    for c in range(NCHUNK):
        k = c % 3
        j = c % 2
        for cp in gather_cps(c, k):
            cp.wait()
        pos_cp(c, j).wait()
        compute_chunk(toks[k], poss[j])
        out_cp(c, k).start()
        if c + 2 < NCHUNK:
            k2 = (c + 2) % 3
            if c - 1 >= 0:
                out_cp(c - 1, k2).wait()
            for cp in gather_cps(c + 2, k2):
                cp.start()
            pos_cp(c + 2, j).start()

    for c in range(NCHUNK - 3, NCHUNK):
        out_cp(c, c % 3).wait()

    for bi in range(B):
        pltpu.sync_copy(acc_v.at[bi], part_hbm.at[bi, wid])


def _ctx_body(part_ref, w_ref, b_ref, out_ref):
    mean = jnp.sum(part_ref[...], axis=1) * (1.0 / S)
    out_ref[...] = (
        jnp.dot(mean, w_ref[...], preferred_element_type=jnp.float32)
        + b_ref[...]
    )


_ctx_proj = pl.pallas_call(
    _ctx_body,
    out_shape=jax.ShapeDtypeStruct((B, D), jnp.float32),
)


def kernel(input_ids, modality, token_embed, pos_embed, modality_embed, W_ctx, b_ctx):
    ids = input_ids.astype(jnp.int32)
    mod_row = lax.dynamic_index_in_dim(
        modality_embed, modality, axis=0, keepdims=False
    )
    emb, part = _sc_embed(ids, token_embed, pos_embed, mod_row)
    context = _ctx_proj(part, W_ctx, b_ctx.reshape(1, D))
    return emb, context


# speedup vs baseline: 3.9918x; 1.0230x over previous
import functools
import math

import jax
import jax.numpy as jnp
from jax import lax
from jax.experimental import pallas as pl
from jax.experimental.pallas import tpu as pltpu
from jax.experimental.pallas import tpu_sc as plsc

B = 4
S = 4096
D = 1024
SCALE = math.sqrt(D)

NC = 2
NS = 16
NW = NC * NS
POS_PER_W = S // NW
CH = 8
NCHUNK = POS_PER_W // CH
G = D // 16

_mesh = plsc.VectorSubcoreMesh(
    core_axis_name="c", subcore_axis_name="s", num_cores=NC, num_subcores=NS
)


@functools.partial(
    pl.kernel,
    out_type=(
        jax.ShapeDtypeStruct((B, S, D), jnp.float32),
        jax.ShapeDtypeStruct((B, NW, D), jnp.float32),
    ),
    mesh=_mesh,
    scratch_types=[
        pltpu.VMEM((NCHUNK * B * CH,), jnp.int32),
        pltpu.VMEM((B * CH, D), jnp.float32),
        pltpu.VMEM((B * CH, D), jnp.float32),
        pltpu.VMEM((B * CH, D), jnp.float32),
        pltpu.VMEM((CH, D), jnp.float32),
        pltpu.VMEM((CH, D), jnp.float32),
        pltpu.VMEM((D,), jnp.float32),
        pltpu.VMEM((B, D), jnp.float32),
        pltpu.SemaphoreType.DMA,
        pltpu.SemaphoreType.DMA,
        pltpu.SemaphoreType.DMA,
        pltpu.SemaphoreType.DMA,
        pltpu.SemaphoreType.DMA,
        pltpu.SemaphoreType.DMA,
        pltpu.SemaphoreType.DMA,
        pltpu.SemaphoreType.DMA,
    ],
)
def _sc_embed(ids_hbm, tok_hbm, pos_hbm, mod_hbm,
              out_hbm, part_hbm,
              idx_v, tokA, tokB, tokC, pos0, pos1, mod_v, acc_v,
              gtA, gtB, gtC, gp0, gp1, goA, goB, goC):
    wid = lax.axis_index("s") * NC + lax.axis_index("c")
    seq0 = wid * POS_PER_W

    toks = (tokA, tokB, tokC)
    gts = (gtA, gtB, gtC)
    gos = (goA, goB, goC)
    poss = (pos0, pos1)
    gps = (gp0, gp1)

    pltpu.sync_copy(ids_hbm.at[wid], idx_v)
    pltpu.sync_copy(mod_hbm, mod_v)

    def zero_body(j, _):
        acc_v[j // G, pl.ds((j % G) * 16, 16)] = jnp.zeros((16,), jnp.float32)
        return 0
    lax.fori_loop(0, B * G, zero_body, 0)

    def gather_cps(c, k):
        buf, sem = toks[k], gts[k]
        return [
            pltpu.make_async_copy(
                tok_hbm.at[idx_v.at[pl.ds(c * B * CH, B * CH)]], buf, sem)
        ]

    def pos_cp(c, j):
        return pltpu.make_async_copy(
            pos_hbm.at[pl.ds(seq0 + c * CH, CH)], poss[j], gps[j])

    def out_cps(c, k):
        buf, sem = toks[k], gos[k]
        return [
            pltpu.make_async_copy(
                buf.at[pl.ds(bi * CH, CH)],
                out_hbm.at[bi, pl.ds(seq0 + c * CH, CH)], sem)
            for bi in range(B)
        ]

    def compute_chunk(tok_v, pos_v):
        def g_body(g, _):
            o = g * 16
            m = mod_v[pl.ds(o, 16)]
            parts = [[None, None] for _ in range(B)]
            for r in range(CH):
                p = pos_v[r, pl.ds(o, 16)]
                pm = p + m
                for bi in range(B):
                    t = tok_v[bi * CH + r, pl.ds(o, 16)]
                    e = t * SCALE + pm
                    tok_v[bi * CH + r, pl.ds(o, 16)] = e
                    k = r % 2
                    pb = parts[bi]
                    pb[k] = e if pb[k] is None else pb[k] + e
            for bi in range(B):
                acc_v[bi, pl.ds(o, 16)] = (
                    acc_v[bi, pl.ds(o, 16)] + (parts[bi][0] + parts[bi][1])
                )
            return 0
        lax.fori_loop(0, G, g_body, 0)

    for cp in gather_cps(0, 0):
        cp.start()
    pos_cp(0, 0).start()
    for cp in gather_cps(1, 1):
        cp.start()
    pos_cp(1, 1).start()

    for c in range(NCHUNK):
        k = c % 3
        j = c % 2
        for cp in gather_cps(c, k):
            cp.wait()
        pos_cp(c, j).wait()
        compute_chunk(toks[k], poss[j])
        for cp in out_cps(c, k):
            cp.start()
        if c + 2 < NCHUNK:
            k2 = (c + 2) % 3
            if c - 1 >= 0:
                for cp in out_cps(c - 1, k2):
                    cp.wait()
            for cp in gather_cps(c + 2, k2):
                cp.start()
            pos_cp(c + 2, j).start()

    for c in range(NCHUNK - 3, NCHUNK):
        for cp in out_cps(c, c % 3):
            cp.wait()

    for bi in range(B):
        pltpu.sync_copy(acc_v.at[bi], part_hbm.at[bi, wid])


def _ctx_body(part_ref, w_ref, b_ref, out_ref):
    mean = jnp.sum(part_ref[...], axis=1) * (1.0 / S)
    out_ref[...] = (
        jnp.dot(mean, w_ref[...], preferred_element_type=jnp.float32)
        + b_ref[...]
    )


_ctx_proj = pl.pallas_call(
    _ctx_body,
    out_shape=jax.ShapeDtypeStruct((B, D), jnp.float32),
)


def kernel(input_ids, modality, token_embed, pos_embed, modality_embed, W_ctx, b_ctx):
    ids = (
        input_ids.astype(jnp.int32)
        .reshape(B, NW, NCHUNK, CH)
        .transpose(1, 2, 0, 3)
        .reshape(NW, NCHUNK * B * CH)
    )
    mod_row = lax.dynamic_index_in_dim(
        modality_embed, modality, axis=0, keepdims=False
    )
    emb, part = _sc_embed(ids, token_embed, pos_embed, mod_row)
    context = _ctx_proj(part, W_ctx, b_ctx.reshape(1, D))
    return emb, context
